# baseline (speedup 1.0000x reference)
import functools

import jax
import jax.numpy as jnp
from jax import lax
from jax.experimental import pallas as pl
from jax.experimental.pallas import tpu as pltpu
from jax.experimental.pallas import tpu_sc as plsc

VOCAB = 100000
EMBED_DIM = 64
BATCH = 16384

_info = plsc.get_sparse_core_info()
_NC, _NS = _info.num_cores, _info.num_subcores
_NW = _NC * _NS
_B_PER_W = BATCH // _NW

_TBLK = 384
_TAIL_BLK = 260
_TAIL = 128
_NTAIL32 = 32
_ALIGNED = _TAIL_BLK * _TBLK + _TAIL
_KMAX = _TAIL_BLK // _NW + 1

_mesh = plsc.VectorSubcoreMesh(core_axis_name="c", subcore_axis_name="s")


def _transpose_block(tin, tout, ncols):
    f_vecs = [16 * j + lax.iota(jnp.int32, 16) for j in range(EMBED_DIM // 16)]

    def group(g, _):
        for r in range(16):
            c = g * 16 + r
            bc = jnp.full((16,), c, dtype=jnp.int32)
            for j in range(EMBED_DIM // 16):
                vec = plsc.load_gather(tin, [f_vecs[j], bc])
                tout[c, pl.ds(j * 16, 16)] = vec
        return ()

    lax.fori_loop(0, ncols // 16, group, ())


@functools.partial(
    pl.kernel,
    mesh=_mesh,
    compiler_params=pltpu.CompilerParams(needs_layout_passes=False),
    out_type=jax.ShapeDtypeStruct((VOCAB, EMBED_DIM), jnp.float32),
    scratch_types=[
        pltpu.VMEM((EMBED_DIM, _TBLK), jnp.float32),
        pltpu.VMEM((EMBED_DIM, _TBLK), jnp.float32),
        pltpu.VMEM((_TBLK, EMBED_DIM), jnp.float32),
        pltpu.SemaphoreType.DMA,
        pltpu.SemaphoreType.DMA,
    ],
)
def _format_table(tab_t_hbm, tail_hbm, out_hbm, tin0, tin1, tout, sem0, sem1):
    wid = lax.axis_index("s") * _NC + lax.axis_index("c")
    tins = (tin0, tin1)
    sems = (sem0, sem1)

    @pl.when(wid == 5)
    def _():
        pltpu.sync_copy(
            tail_hbm,
            out_hbm.at[pl.ds(_ALIGNED, _NTAIL32)],
        )

    b0 = wid
    pltpu.async_copy(tab_t_hbm.at[:, pl.ds(b0 * _TBLK, _TBLK)], tin0, sem0)
    for k in range(_KMAX):
        b = wid + k * _NW
        tin = tins[k % 2]
        nxt = tins[(k + 1) % 2]
        sem = sems[k % 2]
        nsem = sems[(k + 1) % 2]

        @pl.when(b < _TAIL_BLK)
        def _():
            pltpu.make_async_copy(
                tab_t_hbm.at[:, pl.ds(0, _TBLK)], tin, sem
            ).wait()

        @pl.when(b == _TAIL_BLK)
        def _():
            pltpu.make_async_copy(
                tab_t_hbm.at[:, pl.ds(0, _TAIL)],
                tin.at[:, pl.ds(0, _TAIL)],
                sem,
            ).wait()

        nb = b + _NW

        @pl.when(nb < _TAIL_BLK)
        def _():
            pltpu.async_copy(
                tab_t_hbm.at[:, pl.ds(nb * _TBLK, _TBLK)], nxt, nsem
            )

        @pl.when(nb == _TAIL_BLK)
        def _():
            pltpu.async_copy(
                tab_t_hbm.at[:, pl.ds(_TAIL_BLK * _TBLK, _TAIL)],
                nxt.at[:, pl.ds(0, _TAIL)],
                nsem,
            )

        @pl.when(b < _TAIL_BLK)
        def _():
            _transpose_block(tin, tout, _TBLK)
            pltpu.sync_copy(tout, out_hbm.at[pl.ds(b * _TBLK, _TBLK)])

        @pl.when(b == _TAIL_BLK)
        def _():
            _transpose_block(tin, tout, _TAIL)
            pltpu.sync_copy(
                tout.at[pl.ds(0, _TAIL)],
                out_hbm.at[pl.ds(_TAIL_BLK * _TBLK, _TAIL)],
            )


@functools.partial(
    pl.kernel,
    mesh=_mesh,
    out_type=jax.ShapeDtypeStruct((BATCH, EMBED_DIM), jnp.float32),
    scratch_types=[
        pltpu.VMEM((_B_PER_W,), jnp.int32),
        pltpu.VMEM((_B_PER_W, EMBED_DIM), jnp.float32),
        pltpu.SemaphoreType.DMA,
    ],
)
def _embedding_gather(table_hbm, idx_hbm, out_hbm, idx_v, rows_v, sem):
    wid = lax.axis_index("s") * _NC + lax.axis_index("c")
    base = wid * _B_PER_W
    pltpu.sync_copy(idx_hbm.at[pl.ds(base, _B_PER_W)], idx_v)

    def body(g, _):
        vec = idx_v[pl.ds(g * 16, 16)]
        for l in range(16):
            pltpu.async_copy(
                table_hbm.at[pl.ds(vec[l], 1)],
                rows_v.at[pl.ds(g * 16 + l, 1)],
                sem,
            )
        return ()

    lax.fori_loop(0, _B_PER_W // 16, body, ())
    pltpu.make_async_copy(
        table_hbm.at[pl.ds(0, _B_PER_W)], rows_v, sem
    ).wait()
    pltpu.sync_copy(rows_v, out_hbm.at[pl.ds(base, _B_PER_W)])


def kernel(drug_ids, table):
    table_rows = _format_table(table.T, table[_ALIGNED:])
    return _embedding_gather(table_rows, drug_ids.astype(jnp.int32))


# speedup vs baseline: 1.4895x; 1.4895x over previous
import functools

import jax
import jax.numpy as jnp
from jax import lax
from jax.experimental import pallas as pl
from jax.experimental.pallas import tpu as pltpu
from jax.experimental.pallas import tpu_sc as plsc

VOCAB = 100000
EMBED_DIM = 64
BATCH = 16384

_info = plsc.get_sparse_core_info()
_NC, _NS = _info.num_cores, _info.num_subcores
_NW = _NC * _NS
_B_PER_W = BATCH // _NW

_TBLK = 384
_TAIL_BLK = 260
_TAIL = 128
_NTAIL32 = 32
_ALIGNED = _TAIL_BLK * _TBLK + _TAIL
_KMAX = _TAIL_BLK // _NW + 1

_mesh = plsc.VectorSubcoreMesh(core_axis_name="c", subcore_axis_name="s")


def _transpose_block(tin, tout, ncols):
    f_vecs = [16 * j + lax.iota(jnp.int32, 16) for j in range(EMBED_DIM // 16)]

    @plsc.parallel_loop(0, ncols // 16)
    def group(g):
        for r in range(16):
            c = g * 16 + r
            bc = jnp.full((16,), c, dtype=jnp.int32)
            for j in range(EMBED_DIM // 16):
                vec = plsc.load_gather(tin, [f_vecs[j], bc])
                tout[c, pl.ds(j * 16, 16)] = vec


@functools.partial(
    pl.kernel,
    mesh=_mesh,
    compiler_params=pltpu.CompilerParams(needs_layout_passes=False),
    out_type=jax.ShapeDtypeStruct((VOCAB, EMBED_DIM), jnp.float32),
    scratch_types=[
        pltpu.VMEM((EMBED_DIM, _TBLK), jnp.float32),
        pltpu.VMEM((EMBED_DIM, _TBLK), jnp.float32),
        pltpu.VMEM((_TBLK, EMBED_DIM), jnp.float32),
        pltpu.SemaphoreType.DMA,
        pltpu.SemaphoreType.DMA,
    ],
)
def _format_table(tab_t_hbm, tail_hbm, out_hbm, tin0, tin1, tout, sem0, sem1):
    wid = lax.axis_index("s") * _NC + lax.axis_index("c")
    tins = (tin0, tin1)
    sems = (sem0, sem1)

    @pl.when(wid == 5)
    def _():
        pltpu.sync_copy(
            tail_hbm,
            out_hbm.at[pl.ds(_ALIGNED, _NTAIL32)],
        )

    b0 = wid
    pltpu.async_copy(tab_t_hbm.at[:, pl.ds(b0 * _TBLK, _TBLK)], tin0, sem0)
    for k in range(_KMAX):
        b = wid + k * _NW
        tin = tins[k % 2]
        nxt = tins[(k + 1) % 2]
        sem = sems[k % 2]
        nsem = sems[(k + 1) % 2]

        @pl.when(b < _TAIL_BLK)
        def _():
            pltpu.make_async_copy(
                tab_t_hbm.at[:, pl.ds(0, _TBLK)], tin, sem
            ).wait()

        @pl.when(b == _TAIL_BLK)
        def _():
            pltpu.make_async_copy(
                tab_t_hbm.at[:, pl.ds(0, _TAIL)],
                tin.at[:, pl.ds(0, _TAIL)],
                sem,
            ).wait()

        nb = b + _NW

        @pl.when(nb < _TAIL_BLK)
        def _():
            pltpu.async_copy(
                tab_t_hbm.at[:, pl.ds(nb * _TBLK, _TBLK)], nxt, nsem
            )

        @pl.when(nb == _TAIL_BLK)
        def _():
            pltpu.async_copy(
                tab_t_hbm.at[:, pl.ds(_TAIL_BLK * _TBLK, _TAIL)],
                nxt.at[:, pl.ds(0, _TAIL)],
                nsem,
            )

        @pl.when(b <= _TAIL_BLK)
        def _():
            _transpose_block(tin, tout, _TBLK)

        @pl.when(b < _TAIL_BLK)
        def _():
            pltpu.sync_copy(tout, out_hbm.at[pl.ds(b * _TBLK, _TBLK)])

        @pl.when(b == _TAIL_BLK)
        def _():
            pltpu.sync_copy(
                tout.at[pl.ds(0, _TAIL)],
                out_hbm.at[pl.ds(_TAIL_BLK * _TBLK, _TAIL)],
            )


@functools.partial(
    pl.kernel,
    mesh=_mesh,
    out_type=jax.ShapeDtypeStruct((BATCH, EMBED_DIM), jnp.float32),
    scratch_types=[
        pltpu.VMEM((_B_PER_W,), jnp.int32),
        pltpu.VMEM((_B_PER_W, EMBED_DIM), jnp.float32),
        pltpu.SemaphoreType.DMA,
    ],
)
def _embedding_gather(table_hbm, idx_hbm, out_hbm, idx_v, rows_v, sem):
    wid = lax.axis_index("s") * _NC + lax.axis_index("c")
    base = wid * _B_PER_W
    pltpu.sync_copy(idx_hbm.at[pl.ds(base, _B_PER_W)], idx_v)

    def body(g, _):
        vec = idx_v[pl.ds(g * 16, 16)]
        for l in range(16):
            pltpu.async_copy(
                table_hbm.at[pl.ds(vec[l], 1)],
                rows_v.at[pl.ds(g * 16 + l, 1)],
                sem,
            )
        return ()

    lax.fori_loop(0, _B_PER_W // 16, body, ())
    pltpu.make_async_copy(
        table_hbm.at[pl.ds(0, _B_PER_W)], rows_v, sem
    ).wait()
    pltpu.sync_copy(rows_v, out_hbm.at[pl.ds(base, _B_PER_W)])


def kernel(drug_ids, table):
    table_rows = _format_table(table.T, table[_ALIGNED:])
    return _embedding_gather(table_rows, drug_ids.astype(jnp.int32))


# speedup vs baseline: 2.9697x; 1.9937x over previous
import functools

import jax
import jax.numpy as jnp
from jax import lax
from jax.experimental import pallas as pl
from jax.experimental.pallas import tpu as pltpu
from jax.experimental.pallas import tpu_sc as plsc

VOCAB = 100000
EMBED_DIM = 64
BATCH = 16384

_info = plsc.get_sparse_core_info()
_NC, _NS = _info.num_cores, _info.num_subcores
_NW = _NC * _NS
_B_PER_W = BATCH // _NW

_TBLK = 2048
_TGRID = -(-VOCAB // _TBLK)

_mesh = plsc.VectorSubcoreMesh(core_axis_name="c", subcore_axis_name="s")


def _transpose_body(x_ref, o_ref):
    o_ref[...] = x_ref[...].T


_tc_transpose = pl.pallas_call(
    _transpose_body,
    grid=(_TGRID,),
    in_specs=[pl.BlockSpec((EMBED_DIM, _TBLK), lambda i: (0, i))],
    out_specs=pl.BlockSpec((_TBLK, EMBED_DIM), lambda i: (i, 0)),
    out_shape=jax.ShapeDtypeStruct((VOCAB, EMBED_DIM), jnp.float32),
)


@functools.partial(
    pl.kernel,
    mesh=_mesh,
    out_type=jax.ShapeDtypeStruct((BATCH, EMBED_DIM), jnp.float32),
    scratch_types=[
        pltpu.VMEM((_B_PER_W,), jnp.int32),
        pltpu.VMEM((_B_PER_W, EMBED_DIM), jnp.float32),
        pltpu.SemaphoreType.DMA,
    ],
)
def _embedding_gather(table_hbm, idx_hbm, out_hbm, idx_v, rows_v, sem):
    wid = lax.axis_index("s") * _NC + lax.axis_index("c")
    base = wid * _B_PER_W
    pltpu.sync_copy(idx_hbm.at[pl.ds(base, _B_PER_W)], idx_v)

    def body(g, _):
        vec = idx_v[pl.ds(g * 16, 16)]
        for l in range(16):
            pltpu.async_copy(
                table_hbm.at[pl.ds(vec[l], 1)],
                rows_v.at[pl.ds(g * 16 + l, 1)],
                sem,
            )
        return ()

    lax.fori_loop(0, _B_PER_W // 16, body, ())
    pltpu.make_async_copy(
        table_hbm.at[pl.ds(0, _B_PER_W)], rows_v, sem
    ).wait()
    pltpu.sync_copy(rows_v, out_hbm.at[pl.ds(base, _B_PER_W)])


def kernel(drug_ids, table):
    table_rows = _tc_transpose(table.T)
    return _embedding_gather(table_rows, drug_ids.astype(jnp.int32))
